# baseline (speedup 1.0000x reference)
import functools

import jax
import jax.numpy as jnp
from jax import lax
from jax.experimental import pallas as pl
from jax.experimental.pallas import tpu as pltpu
from jax.experimental.pallas import tpu_sc as plsc

_N = 10000
_T = 16
_E = 160000
_B = 64
_H = 128
_HG = 128
_RB = 1024
_NP = 10240
_NBLK = _NP // _RB
_TILES = 16
_G = 128
_NB_E = 84
_EPT = _NB_E * _G
_EP = _TILES * _EPT
_RT = _NP // _TILES
_W = 32
_DR = _NP // _W
_CAP = 5888
_NB_B = _CAP // _G


def _lstm_xw_body(x_ref, wih_f_ref, whhT_f_ref, b_f_ref,
                  wih_r_ref, whhT_r_ref, b_r_ref,
                  wgf_ref, wgr_ref, xw_ref):
    x = x_ref[...]
    whhT_f = whhT_f_ref[...]
    whhT_r = whhT_r_ref[...]
    wih_f = wih_f_ref[...]
    wih_r = wih_r_ref[...]
    b_f = b_f_ref[...]
    b_r = b_r_ref[...]
    wgf = wgf_ref[...]
    wgr = wgr_ref[...]
    R = x.shape[0]

    def cell(h, c, xt, whhT, wih, b):
        g = (jnp.dot(h, whhT, preferred_element_type=jnp.float32)
             + xt[:, None] * wih + b)
        i = jax.nn.sigmoid(g[:, :_H])
        f = jax.nn.sigmoid(g[:, _H:2 * _H])
        gg = jnp.tanh(g[:, 2 * _H:3 * _H])
        o = jax.nn.sigmoid(g[:, 3 * _H:])
        c = f * c + i * gg
        h = o * jnp.tanh(c)
        return h, c

    h = jnp.zeros((R, _H), jnp.float32)
    c = jnp.zeros((R, _H), jnp.float32)
    hfs = []
    for t in range(_T):
        h, c = cell(h, c, x[:, t], whhT_f, wih_f, b_f)
        hfs.append(h)
    h = jnp.zeros((R, _H), jnp.float32)
    c = jnp.zeros((R, _H), jnp.float32)
    for t in range(_T - 1, -1, -1):
        h, c = cell(h, c, x[:, t], whhT_r, wih_r, b_r)
        xw_ref[t] = (jnp.dot(hfs[t], wgf, preferred_element_type=jnp.float32)
                     + jnp.dot(h, wgr, preferred_element_type=jnp.float32))


def _lstm_xw(x, Wih_f, Whh_f, bih_f, bhh_f, Wih_r, Whh_r, bih_r, bhh_r, W_gcn):
    xp = jnp.pad(x, ((0, _NP - _N), (0, 0)))
    args = (
        xp,
        Wih_f[:, 0][None, :], Whh_f.T, (bih_f + bhh_f)[None, :],
        Wih_r[:, 0][None, :], Whh_r.T, (bih_r + bhh_r)[None, :],
        W_gcn[:_H], W_gcn[_H:],
    )
    full = lambda s: pl.BlockSpec(s, lambda i: (0,) * len(s))
    return pl.pallas_call(
        _lstm_xw_body,
        grid=(_NBLK,),
        in_specs=[
            pl.BlockSpec((_RB, _T), lambda i: (i, 0)),
            full((1, 4 * _H)), full((_H, 4 * _H)), full((1, 4 * _H)),
            full((1, 4 * _H)), full((_H, 4 * _H)), full((1, 4 * _H)),
            full((_H, _HG)), full((_H, _HG)),
        ],
        out_specs=pl.BlockSpec((_T, _RB, _HG), lambda i: (0, i, 0)),
        out_shape=jax.ShapeDtypeStruct((_T, _NP, _HG), jnp.float32),
        compiler_params=pltpu.CompilerParams(
            dimension_semantics=("arbitrary",)),
    )(*args)


def _sc_body(xw_hbm, rcol_hbm, rew_hbm, brow_hbm, bcol_hbm, bew_hbm, acc_hbm,
             row_t, col_t, ew_t, norm_t, gidx_t, dbuf, dvbuf,
             dinv_t, acc_t, buf0, buf1, zbuf, deg_sh, dinv_sh, sem0, sem1):
    c = lax.axis_index("c")
    s = lax.axis_index("s")
    wid = c * _TILES + s
    f32 = jnp.float32
    zeros16 = jnp.zeros((16,), f32)
    bufs = (buf0, buf1)
    sems = (sem0, sem1)

    def zero_zbuf(i, carry):
        for k in range(8):
            zbuf[i, pl.ds(k * 16, 16)] = zeros16
        return carry
    lax.fori_loop(0, 32, zero_zbuf, 0)

    def zero_dbuf(i, carry):
        dbuf[pl.ds(i * 16, 16)] = zeros16
        return carry
    lax.fori_loop(0, _RT // 16, zero_dbuf, 0)

    pltpu.sync_copy(dbuf, deg_sh.at[pl.ds(s * _RT, _RT)])
    plsc.subcore_barrier()

    for piece in range(2):
        pltpu.sync_copy(rcol_hbm.at[s, piece], col_t.at[pl.ds(0, 42)])
        pltpu.sync_copy(rew_hbm.at[s, piece], ew_t.at[pl.ds(0, 42)])

        def deg_j(j, carry):
            pltpu.sync_copy(ew_t.at[j], deg_sh.at[col_t.at[j]], add=True)
            return carry
        lax.fori_loop(0, 42, deg_j, 0)
    plsc.subcore_barrier()

    pltpu.sync_copy(deg_sh.at[pl.ds(s * _RT, _RT)], dbuf)

    def newt(i, carry):
        v = dbuf[pl.ds(i * 16, 16)]
        yi = jnp.int32(0x5F3759DF) - (
            lax.bitcast_convert_type(v, jnp.int32) >> 1)
        y = lax.bitcast_convert_type(yi, f32)
        for _ in range(3):
            y = y * (1.5 - 0.5 * v * y * y)
        dvbuf[pl.ds(i * 16, 16)] = y
        return carry
    lax.fori_loop(0, _RT // 16, newt, 0)
    pltpu.sync_copy(dvbuf, dinv_sh.at[pl.ds(s * _RT, _RT)])
    plsc.subcore_barrier()
    pltpu.sync_copy(dinv_sh, dinv_t)

    pltpu.sync_copy(brow_hbm.at[wid], row_t)
    pltpu.sync_copy(bcol_hbm.at[wid], col_t)
    pltpu.sync_copy(bew_hbm.at[wid], ew_t)

    lo_w = wid * _DR

    def norm_j(j, carry):
        for k in range(8):
            rv = row_t[j, pl.ds(k * 16, 16)]
            cv = col_t[j, pl.ds(k * 16, 16)]
            ev = ew_t[j, pl.ds(k * 16, 16)]
            nv = (plsc.load_gather(dinv_t, [rv]) * ev
                  * plsc.load_gather(dinv_t, [cv]))
            norm_t[j, pl.ds(k * 16, 16)] = nv
            col_t[j, pl.ds(k * 16, 16)] = cv - lo_w
        return carry
    lax.fori_loop(0, _NB_B, norm_j, 0)

    def t_body(t, carry):
        def zcp(m, carry2):
            for k in range(8):
                acc_t[m, pl.ds(k * 16, 16)] = zeros16
            return carry2
        lax.fori_loop(0, _DR, zcp, 0)

        base = t * _NP

        def gj(j, carry2):
            for k in range(8):
                gidx_t[j, pl.ds(k * 16, 16)] = (
                    row_t[j, pl.ds(k * 16, 16)] + base)
            return carry2
        lax.fori_loop(0, _NB_B, gj, 0)

        pltpu.async_copy(xw_hbm.at[gidx_t.at[0]], buf0, sem0)

        def pair(jj, carry2):
            j = jj * 2
            for b in range(2):
                jb = j + b
                nb = jb + 1

                @pl.when(nb < _NB_B)
                def _():
                    pltpu.async_copy(xw_hbm.at[gidx_t.at[nb]],
                                     bufs[(b + 1) % 2], sems[(b + 1) % 2])

                pltpu.make_async_copy(xw_hbm.at[gidx_t.at[jb]],
                                      bufs[b], sems[b]).wait()
                buf = bufs[b]

                def accum_q(qq, carry3):
                    nv = norm_t[jb, pl.ds(qq * 16, 16)]
                    lv = col_t[jb, pl.ds(qq * 16, 16)]
                    for l in range(16):
                        e = qq * 16 + l
                        sv = nv[l]
                        lr = lv[l]
                        for k in range(8):
                            acc_t[lr, pl.ds(k * 16, 16)] = (
                                acc_t[lr, pl.ds(k * 16, 16)]
                                + buf[e, pl.ds(k * 16, 16)] * sv)
                    return carry3
                lax.fori_loop(0, _G // 16, accum_q, 0)
            return carry2
        lax.fori_loop(0, _NB_B // 2, pair, 0)

        pltpu.sync_copy(acc_t, acc_hbm.at[t, pl.ds(lo_w, _DR)])
        return carry
    lax.fori_loop(0, _T, t_body, 0)


def _sc_prop(xwflat, rcol4, rew4, brow3, bcol3, bew3):
    mesh = plsc.VectorSubcoreMesh(core_axis_name="c", subcore_axis_name="s",
                                  num_cores=2, num_subcores=_TILES)
    f = pl.kernel(
        _sc_body, mesh=mesh,
        compiler_params=pltpu.CompilerParams(needs_layout_passes=False),
        out_type=jax.ShapeDtypeStruct((_T, _NP, _HG), jnp.float32),
        scratch_types=[
            pltpu.VMEM((_NB_B, _G), jnp.int32),
            pltpu.VMEM((_NB_B, _G), jnp.int32),
            pltpu.VMEM((_NB_B, _G), jnp.float32),
            pltpu.VMEM((_NB_B, _G), jnp.float32),
            pltpu.VMEM((_NB_B, _G), jnp.int32),
            pltpu.VMEM((_RT,), jnp.float32),
            pltpu.VMEM((_RT,), jnp.float32),
            pltpu.VMEM((_NP,), jnp.float32),
            pltpu.VMEM((_DR, _HG), jnp.float32),
            pltpu.VMEM((_G, _HG), jnp.float32),
            pltpu.VMEM((_G, _HG), jnp.float32),
            pltpu.VMEM((32, _HG), jnp.float32),
            pltpu.VMEM_SHARED((_NP,), jnp.float32),
            pltpu.VMEM_SHARED((_NP,), jnp.float32),
            pltpu.SemaphoreType.DMA,
            pltpu.SemaphoreType.DMA,
        ],
    )
    return f(xwflat, rcol4, rew4, brow3, bcol3, bew3)


def _tail_body(acc_ref, batch_ref, bg_ref, wcls_ref, bcls_ref, out_ref,
               pooled, counts):
    i = pl.program_id(0)

    @pl.when(i == 0)
    def _():
        pooled[...] = jnp.zeros_like(pooled)
        counts[...] = jnp.zeros_like(counts)
        out_ref[...] = jnp.zeros_like(out_ref)

    acc = acc_ref[...].astype(jnp.float32)
    bg = bg_ref[...][0:1, :]
    feats = jax.nn.relu(acc + bg[None])
    node = jnp.mean(feats, axis=0)
    bvec = batch_ref[...].reshape(1, _RB)
    ohT = (jax.lax.broadcasted_iota(jnp.int32, (_B, _RB), 0)
           == bvec).astype(jnp.float32)
    pooled[...] += jnp.dot(ohT, node, preferred_element_type=jnp.float32)
    counts[...] += jnp.broadcast_to(
        jnp.sum(ohT, axis=1, keepdims=True), (_B, _HG))

    @pl.when(i == _NBLK - 1)
    def _():
        g = pooled[...] / jnp.maximum(counts[...], 1.0)
        out_ref[...] = (jnp.dot(g, wcls_ref[...],
                                preferred_element_type=jnp.float32)
                        + bcls_ref[...][0:1, :])


def _tail(acc, batch3, bg8, W_cls, bcls8):
    full = lambda s: pl.BlockSpec(s, lambda i: (0,) * len(s))
    return pl.pallas_call(
        _tail_body,
        grid=(_NBLK,),
        in_specs=[
            pl.BlockSpec((_T, _RB, _HG), lambda i: (0, i, 0)),
            pl.BlockSpec((1, 1, _RB), lambda i: (i, 0, 0)),
            full((8, _HG)), full((_HG, 3)), full((8, 3)),
        ],
        out_specs=full((_B, 3)),
        out_shape=jax.ShapeDtypeStruct((_B, 3), jnp.float32),
        scratch_shapes=[
            pltpu.VMEM((_B, _HG), jnp.float32),
            pltpu.VMEM((_B, _HG), jnp.float32),
        ],
        compiler_params=pltpu.CompilerParams(
            dimension_semantics=("arbitrary",)),
    )(acc, batch3, bg8, W_cls, bcls8)


def kernel(x, edge_index, edge_weight, batch,
           Wih_f, Whh_f, bih_f, bhh_f,
           Wih_r, Whh_r, bih_r, bhh_r,
           W_gcn, b_gcn, W_cls, b_cls):
    xw = _lstm_xw(x, Wih_f, Whh_f, bih_f, bhh_f,
                  Wih_r, Whh_r, bih_r, bhh_r, W_gcn)
    xwflat = xw.reshape(_T * _NP, _HG)

    row = edge_index[0]
    col = edge_index[1]
    loop_idx = jnp.arange(_N, dtype=jnp.int32)
    pad = _EP - _E - _N
    row0 = jnp.concatenate([row, loop_idx])
    col0 = jnp.concatenate([col, loop_idx])
    ew0 = jnp.concatenate([edge_weight, jnp.ones((_N,), jnp.float32)])
    cola = jnp.concatenate([col0, jnp.zeros((pad,), jnp.int32)]).reshape(
        _TILES, 2, _EPT // (2 * _G), _G)
    ewa = jnp.concatenate([ew0, jnp.zeros((pad,), jnp.float32)]).reshape(
        _TILES, 2, _EPT // (2 * _G), _G)
    nb = col0 // _DR
    order = jnp.argsort(nb, stable=True)
    nb_s = nb[order]
    offs = jnp.cumsum(jnp.bincount(nb, length=_W)) - jnp.bincount(
        nb, length=_W)
    pos = jnp.arange(row0.shape[0], dtype=jnp.int32) - offs[nb_s]
    dest = nb_s * _CAP + pos
    slots = jnp.arange(_W * _CAP, dtype=jnp.int32)
    browb = jnp.zeros((_W * _CAP,), jnp.int32).at[dest].set(
        row0[order], mode='drop').reshape(_W, _NB_B, _G)
    bcolb = ((slots // _CAP) * _DR).at[dest].set(
        col0[order], mode='drop').reshape(_W, _NB_B, _G)
    bewb = jnp.zeros((_W * _CAP,), jnp.float32).at[dest].set(
        ew0[order], mode='drop').reshape(_W, _NB_B, _G)

    acc = _sc_prop(xwflat, cola, ewa, browb, bcolb, bewb)

    batch3 = jnp.pad(batch, (0, _NP - _N),
                     constant_values=_B).reshape(_NBLK, 1, _RB)
    bg8 = jnp.broadcast_to(b_gcn[None, :], (8, _HG))
    bcls8 = jnp.broadcast_to(b_cls[None, :], (8, 3))
    return _tail(acc, batch3, bg8, W_cls, bcls8)


# speedup vs baseline: 1.0425x; 1.0425x over previous
import functools

import jax
import jax.numpy as jnp
from jax import lax
from jax.experimental import pallas as pl
from jax.experimental.pallas import tpu as pltpu
from jax.experimental.pallas import tpu_sc as plsc

_N = 10000
_T = 16
_E = 160000
_B = 64
_H = 128
_HG = 128
_RB = 1024
_NP = 10240
_NBLK = _NP // _RB
_TILES = 16
_G = 128
_NB_E = 84
_EPT = _NB_E * _G
_EP = _TILES * _EPT
_RT = _NP // _TILES
_W = 32
_DR = _NP // _W
_CAP = 5888
_NB_B = _CAP // _G


def _lstm_xw_body(x_ref, wih_f_ref, whhT_f_ref, b_f_ref,
                  wih_r_ref, whhT_r_ref, b_r_ref,
                  wgf_ref, wgr_ref, xw_ref):
    x = x_ref[...]
    whhT_f = whhT_f_ref[...]
    whhT_r = whhT_r_ref[...]
    wih_f = wih_f_ref[...]
    wih_r = wih_r_ref[...]
    b_f = b_f_ref[...]
    b_r = b_r_ref[...]
    wgf = wgf_ref[...]
    wgr = wgr_ref[...]
    R = x.shape[0]

    def cell(h, c, xt, whhT, wih, b):
        g = (jnp.dot(h, whhT, preferred_element_type=jnp.float32)
             + xt[:, None] * wih + b)
        i = jax.nn.sigmoid(g[:, :_H])
        f = jax.nn.sigmoid(g[:, _H:2 * _H])
        gg = jnp.tanh(g[:, 2 * _H:3 * _H])
        o = jax.nn.sigmoid(g[:, 3 * _H:])
        c = f * c + i * gg
        h = o * jnp.tanh(c)
        return h, c

    h = jnp.zeros((R, _H), jnp.float32)
    c = jnp.zeros((R, _H), jnp.float32)
    hfs = []
    for t in range(_T):
        h, c = cell(h, c, x[:, t], whhT_f, wih_f, b_f)
        hfs.append(h)
    h = jnp.zeros((R, _H), jnp.float32)
    c = jnp.zeros((R, _H), jnp.float32)
    for t in range(_T - 1, -1, -1):
        h, c = cell(h, c, x[:, t], whhT_r, wih_r, b_r)
        xw_ref[t] = (jnp.dot(hfs[t], wgf, preferred_element_type=jnp.float32)
                     + jnp.dot(h, wgr, preferred_element_type=jnp.float32))


def _lstm_xw(x, Wih_f, Whh_f, bih_f, bhh_f, Wih_r, Whh_r, bih_r, bhh_r, W_gcn):
    xp = jnp.pad(x, ((0, _NP - _N), (0, 0)))
    args = (
        xp,
        Wih_f[:, 0][None, :], Whh_f.T, (bih_f + bhh_f)[None, :],
        Wih_r[:, 0][None, :], Whh_r.T, (bih_r + bhh_r)[None, :],
        W_gcn[:_H], W_gcn[_H:],
    )
    full = lambda s: pl.BlockSpec(s, lambda i: (0,) * len(s))
    return pl.pallas_call(
        _lstm_xw_body,
        grid=(_NBLK,),
        in_specs=[
            pl.BlockSpec((_RB, _T), lambda i: (i, 0)),
            full((1, 4 * _H)), full((_H, 4 * _H)), full((1, 4 * _H)),
            full((1, 4 * _H)), full((_H, 4 * _H)), full((1, 4 * _H)),
            full((_H, _HG)), full((_H, _HG)),
        ],
        out_specs=pl.BlockSpec((_T, _RB, _HG), lambda i: (0, i, 0)),
        out_shape=jax.ShapeDtypeStruct((_T, _NP, _HG), jnp.float32),
        compiler_params=pltpu.CompilerParams(
            dimension_semantics=("arbitrary",)),
    )(*args)


def _sc_body(xw_hbm, rcol_hbm, rew_hbm, brow_hbm, bcol_hbm, bew_hbm, acc_hbm,
             row_t, col_t, ew_t, norm_t, gidx_t, dbuf, dvbuf,
             dinv_t, acc_t, buf0, buf1, zbuf, deg_sh, dinv_sh, sem0, sem1):
    c = lax.axis_index("c")
    s = lax.axis_index("s")
    wid = c * _TILES + s
    f32 = jnp.float32
    zeros16 = jnp.zeros((16,), f32)
    bufs = (buf0, buf1)
    sems = (sem0, sem1)
    iota16 = lax.iota(jnp.int32, 16)

    def zero_zbuf(i, carry):
        for k in range(8):
            zbuf[i, pl.ds(k * 16, 16)] = zeros16
        return carry
    lax.fori_loop(0, 32, zero_zbuf, 0)

    def zero_dbuf(i, carry):
        dbuf[pl.ds(i * 16, 16)] = zeros16
        return carry
    lax.fori_loop(0, _RT // 16, zero_dbuf, 0)

    pltpu.sync_copy(dbuf, deg_sh.at[pl.ds(s * _RT, _RT)])
    plsc.subcore_barrier()

    for piece in range(2):
        pltpu.sync_copy(rcol_hbm.at[s, piece], col_t.at[pl.ds(0, 42)])
        pltpu.sync_copy(rew_hbm.at[s, piece], ew_t.at[pl.ds(0, 42)])

        def deg_j(j, carry):
            pltpu.sync_copy(ew_t.at[j], deg_sh.at[col_t.at[j]], add=True)
            return carry
        lax.fori_loop(0, 42, deg_j, 0)
    plsc.subcore_barrier()

    pltpu.sync_copy(deg_sh.at[pl.ds(s * _RT, _RT)], dbuf)

    def newt(i, carry):
        v = dbuf[pl.ds(i * 16, 16)]
        yi = jnp.int32(0x5F3759DF) - (
            lax.bitcast_convert_type(v, jnp.int32) >> 1)
        y = lax.bitcast_convert_type(yi, f32)
        for _ in range(3):
            y = y * (1.5 - 0.5 * v * y * y)
        dvbuf[pl.ds(i * 16, 16)] = y
        return carry
    lax.fori_loop(0, _RT // 16, newt, 0)
    pltpu.sync_copy(dvbuf, dinv_sh.at[pl.ds(s * _RT, _RT)])
    plsc.subcore_barrier()
    pltpu.sync_copy(dinv_sh, dinv_t)

    pltpu.sync_copy(brow_hbm.at[wid], row_t)
    pltpu.sync_copy(bcol_hbm.at[wid], col_t)
    pltpu.sync_copy(bew_hbm.at[wid], ew_t)

    lo_w = wid * _DR

    def norm_j(j, carry):
        for k in range(8):
            rv = row_t[j, pl.ds(k * 16, 16)]
            cv = col_t[j, pl.ds(k * 16, 16)]
            ev = ew_t[j, pl.ds(k * 16, 16)]
            nv = (plsc.load_gather(dinv_t, [rv]) * ev
                  * plsc.load_gather(dinv_t, [cv]))
            norm_t[j, pl.ds(k * 16, 16)] = nv
            col_t[j, pl.ds(k * 16, 16)] = cv - lo_w
        return carry
    lax.fori_loop(0, _NB_B, norm_j, 0)

    def t_body(t, carry):
        def zcp(m, carry2):
            acc_t[pl.ds(m * 16, 16)] = zeros16
            return carry2
        lax.fori_loop(0, _DR * _HG // 16, zcp, 0)

        base = t * _NP

        def gj(j, carry2):
            for k in range(8):
                gidx_t[j, pl.ds(k * 16, 16)] = (
                    row_t[j, pl.ds(k * 16, 16)] + base)
            return carry2
        lax.fori_loop(0, _NB_B, gj, 0)

        pltpu.async_copy(xw_hbm.at[gidx_t.at[0]], buf0, sem0)

        def pair(jj, carry2):
            j = jj * 2
            for b in range(2):
                jb = j + b
                nb = jb + 1

                @pl.when(nb < _NB_B)
                def _():
                    pltpu.async_copy(xw_hbm.at[gidx_t.at[nb]],
                                     bufs[(b + 1) % 2], sems[(b + 1) % 2])

                pltpu.make_async_copy(xw_hbm.at[gidx_t.at[jb]],
                                      bufs[b], sems[b]).wait()
                buf = bufs[b]

                def accum_q(qq, carry3):
                    nv = norm_t[jb, pl.ds(qq * 16, 16)]
                    lv = col_t[jb, pl.ds(qq * 16, 16)]
                    for l in range(16):
                        e = qq * 16 + l
                        sv = nv[l]
                        base = lv[l] * _HG
                        for k in range(8):
                            addr = iota16 + (base + k * 16)
                            plsc.addupdate_scatter(
                                acc_t, [addr],
                                buf[e, pl.ds(k * 16, 16)] * sv)
                    return carry3
                lax.fori_loop(0, _G // 16, accum_q, 0)
            return carry2
        lax.fori_loop(0, _NB_B // 2, pair, 0)

        pltpu.sync_copy(acc_t, acc_hbm.at[t, wid])
        return carry
    lax.fori_loop(0, _T, t_body, 0)


def _sc_prop(xwflat, rcol4, rew4, brow3, bcol3, bew3):
    mesh = plsc.VectorSubcoreMesh(core_axis_name="c", subcore_axis_name="s",
                                  num_cores=2, num_subcores=_TILES)
    f = pl.kernel(
        _sc_body, mesh=mesh,
        compiler_params=pltpu.CompilerParams(needs_layout_passes=False),
        out_type=jax.ShapeDtypeStruct((_T, _W, _DR * _HG), jnp.float32),
        scratch_types=[
            pltpu.VMEM((_NB_B, _G), jnp.int32),
            pltpu.VMEM((_NB_B, _G), jnp.int32),
            pltpu.VMEM((_NB_B, _G), jnp.float32),
            pltpu.VMEM((_NB_B, _G), jnp.float32),
            pltpu.VMEM((_NB_B, _G), jnp.int32),
            pltpu.VMEM((_RT,), jnp.float32),
            pltpu.VMEM((_RT,), jnp.float32),
            pltpu.VMEM((_NP,), jnp.float32),
            pltpu.VMEM((_DR * _HG,), jnp.float32),
            pltpu.VMEM((_G, _HG), jnp.float32),
            pltpu.VMEM((_G, _HG), jnp.float32),
            pltpu.VMEM((32, _HG), jnp.float32),
            pltpu.VMEM_SHARED((_NP,), jnp.float32),
            pltpu.VMEM_SHARED((_NP,), jnp.float32),
            pltpu.SemaphoreType.DMA,
            pltpu.SemaphoreType.DMA,
        ],
    )
    return f(xwflat, rcol4, rew4, brow3, bcol3, bew3)


def _tail_body(acc_ref, batch_ref, bg_ref, wcls_ref, bcls_ref, out_ref,
               pooled, counts):
    i = pl.program_id(0)

    @pl.when(i == 0)
    def _():
        pooled[...] = jnp.zeros_like(pooled)
        counts[...] = jnp.zeros_like(counts)
        out_ref[...] = jnp.zeros_like(out_ref)

    acc = acc_ref[...].astype(jnp.float32)
    bg = bg_ref[...][0:1, :]
    feats = jax.nn.relu(acc + bg[None])
    node = jnp.mean(feats, axis=0)
    bvec = batch_ref[...].reshape(1, _RB)
    ohT = (jax.lax.broadcasted_iota(jnp.int32, (_B, _RB), 0)
           == bvec).astype(jnp.float32)
    pooled[...] += jnp.dot(ohT, node, preferred_element_type=jnp.float32)
    counts[...] += jnp.broadcast_to(
        jnp.sum(ohT, axis=1, keepdims=True), (_B, _HG))

    @pl.when(i == _NBLK - 1)
    def _():
        g = pooled[...] / jnp.maximum(counts[...], 1.0)
        out_ref[...] = (jnp.dot(g, wcls_ref[...],
                                preferred_element_type=jnp.float32)
                        + bcls_ref[...][0:1, :])


def _tail(acc, batch3, bg8, W_cls, bcls8):
    full = lambda s: pl.BlockSpec(s, lambda i: (0,) * len(s))
    return pl.pallas_call(
        _tail_body,
        grid=(_NBLK,),
        in_specs=[
            pl.BlockSpec((_T, _RB, _HG), lambda i: (0, i, 0)),
            pl.BlockSpec((1, 1, _RB), lambda i: (i, 0, 0)),
            full((8, _HG)), full((_HG, 3)), full((8, 3)),
        ],
        out_specs=full((_B, 3)),
        out_shape=jax.ShapeDtypeStruct((_B, 3), jnp.float32),
        scratch_shapes=[
            pltpu.VMEM((_B, _HG), jnp.float32),
            pltpu.VMEM((_B, _HG), jnp.float32),
        ],
        compiler_params=pltpu.CompilerParams(
            dimension_semantics=("arbitrary",)),
    )(acc, batch3, bg8, W_cls, bcls8)


def kernel(x, edge_index, edge_weight, batch,
           Wih_f, Whh_f, bih_f, bhh_f,
           Wih_r, Whh_r, bih_r, bhh_r,
           W_gcn, b_gcn, W_cls, b_cls):
    xw = _lstm_xw(x, Wih_f, Whh_f, bih_f, bhh_f,
                  Wih_r, Whh_r, bih_r, bhh_r, W_gcn)
    xwflat = xw.reshape(_T * _NP, _HG)

    row = edge_index[0]
    col = edge_index[1]
    loop_idx = jnp.arange(_N, dtype=jnp.int32)
    pad = _EP - _E - _N
    row0 = jnp.concatenate([row, loop_idx])
    col0 = jnp.concatenate([col, loop_idx])
    ew0 = jnp.concatenate([edge_weight, jnp.ones((_N,), jnp.float32)])
    cola = jnp.concatenate([col0, jnp.zeros((pad,), jnp.int32)]).reshape(
        _TILES, 2, _EPT // (2 * _G), _G)
    ewa = jnp.concatenate([ew0, jnp.zeros((pad,), jnp.float32)]).reshape(
        _TILES, 2, _EPT // (2 * _G), _G)
    nb = col0 // _DR
    order = jnp.argsort(nb, stable=True)
    nb_s = nb[order]
    offs = jnp.cumsum(jnp.bincount(nb, length=_W)) - jnp.bincount(
        nb, length=_W)
    pos = jnp.arange(row0.shape[0], dtype=jnp.int32) - offs[nb_s]
    dest = nb_s * _CAP + pos
    slots = jnp.arange(_W * _CAP, dtype=jnp.int32)
    browb = jnp.zeros((_W * _CAP,), jnp.int32).at[dest].set(
        row0[order], mode='drop').reshape(_W, _NB_B, _G)
    bcolb = ((slots // _CAP) * _DR).at[dest].set(
        col0[order], mode='drop').reshape(_W, _NB_B, _G)
    bewb = jnp.zeros((_W * _CAP,), jnp.float32).at[dest].set(
        ew0[order], mode='drop').reshape(_W, _NB_B, _G)

    acc = _sc_prop(xwflat, cola, ewa, browb, bcolb, bewb).reshape(
        _T, _NP, _HG)

    batch3 = jnp.pad(batch, (0, _NP - _N),
                     constant_values=_B).reshape(_NBLK, 1, _RB)
    bg8 = jnp.broadcast_to(b_gcn[None, :], (8, _HG))
    bcls8 = jnp.broadcast_to(b_cls[None, :], (8, 3))
    return _tail(acc, batch3, bg8, W_cls, bcls8)


# speedup vs baseline: 1.0429x; 1.0005x over previous
import functools

import jax
import jax.numpy as jnp
from jax import lax
from jax.experimental import pallas as pl
from jax.experimental.pallas import tpu as pltpu
from jax.experimental.pallas import tpu_sc as plsc

_N = 10000
_T = 16
_E = 160000
_B = 64
_H = 128
_HG = 128
_RB = 1024
_NP = 10240
_NBLK = _NP // _RB
_TILES = 16
_G = 128
_NB_E = 84
_EPT = _NB_E * _G
_EP = _TILES * _EPT
_RT = _NP // _TILES
_W = 32
_DR = _NP // _W
_CAP = 5888
_NB_B = _CAP // _G


def _lstm_xw_body(x_ref, wih_f_ref, whhT_f_ref, b_f_ref,
                  wih_r_ref, whhT_r_ref, b_r_ref,
                  wgf_ref, wgr_ref, xw_ref):
    x = x_ref[...]
    whhT_f = whhT_f_ref[...]
    whhT_r = whhT_r_ref[...]
    wih_f = wih_f_ref[...]
    wih_r = wih_r_ref[...]
    b_f = b_f_ref[...]
    b_r = b_r_ref[...]
    wgf = wgf_ref[...]
    wgr = wgr_ref[...]
    R = x.shape[0]

    def cell(h, c, xt, whhT, wih, b):
        g = (jnp.dot(h, whhT, preferred_element_type=jnp.float32)
             + xt[:, None] * wih + b)
        i = jax.nn.sigmoid(g[:, :_H])
        f = jax.nn.sigmoid(g[:, _H:2 * _H])
        gg = jnp.tanh(g[:, 2 * _H:3 * _H])
        o = jax.nn.sigmoid(g[:, 3 * _H:])
        c = f * c + i * gg
        h = o * jnp.tanh(c)
        return h, c

    h = jnp.zeros((R, _H), jnp.float32)
    c = jnp.zeros((R, _H), jnp.float32)
    hfs = []
    for t in range(_T):
        h, c = cell(h, c, x[:, t], whhT_f, wih_f, b_f)
        hfs.append(h)
    h = jnp.zeros((R, _H), jnp.float32)
    c = jnp.zeros((R, _H), jnp.float32)
    for t in range(_T - 1, -1, -1):
        h, c = cell(h, c, x[:, t], whhT_r, wih_r, b_r)
        xw_ref[t] = (jnp.dot(hfs[t], wgf, preferred_element_type=jnp.float32)
                     + jnp.dot(h, wgr, preferred_element_type=jnp.float32))


def _lstm_xw(x, Wih_f, Whh_f, bih_f, bhh_f, Wih_r, Whh_r, bih_r, bhh_r, W_gcn):
    xp = jnp.pad(x, ((0, _NP - _N), (0, 0)))
    args = (
        xp,
        Wih_f[:, 0][None, :], Whh_f.T, (bih_f + bhh_f)[None, :],
        Wih_r[:, 0][None, :], Whh_r.T, (bih_r + bhh_r)[None, :],
        W_gcn[:_H], W_gcn[_H:],
    )
    full = lambda s: pl.BlockSpec(s, lambda i: (0,) * len(s))
    return pl.pallas_call(
        _lstm_xw_body,
        grid=(_NBLK,),
        in_specs=[
            pl.BlockSpec((_RB, _T), lambda i: (i, 0)),
            full((1, 4 * _H)), full((_H, 4 * _H)), full((1, 4 * _H)),
            full((1, 4 * _H)), full((_H, 4 * _H)), full((1, 4 * _H)),
            full((_H, _HG)), full((_H, _HG)),
        ],
        out_specs=pl.BlockSpec((_T, _RB, _HG), lambda i: (0, i, 0)),
        out_shape=jax.ShapeDtypeStruct((_T, _NP, _HG), jnp.float32),
        compiler_params=pltpu.CompilerParams(
            dimension_semantics=("arbitrary",)),
    )(*args)


def _sc_body(xw_hbm, rcol_hbm, rew_hbm, brow_hbm, bcol_hbm, bew_hbm, acc_hbm,
             row_t, col_t, ew_t, norm_t, gidx_t, dbuf, dvbuf,
             dinv_t, acc_t, buf0, buf1, zbuf, deg_sh, dinv_sh, sem0, sem1):
    c = lax.axis_index("c")
    s = lax.axis_index("s")
    wid = c * _TILES + s
    f32 = jnp.float32
    zeros16 = jnp.zeros((16,), f32)
    bufs = (buf0, buf1)
    sems = (sem0, sem1)
    iota16 = lax.iota(jnp.int32, 16)

    def zero_zbuf(i, carry):
        for k in range(8):
            zbuf[i, pl.ds(k * 16, 16)] = zeros16
        return carry
    lax.fori_loop(0, 32, zero_zbuf, 0)

    def zero_dbuf(i, carry):
        dbuf[pl.ds(i * 16, 16)] = zeros16
        return carry
    lax.fori_loop(0, _RT // 16, zero_dbuf, 0)

    pltpu.sync_copy(dbuf, deg_sh.at[pl.ds(s * _RT, _RT)])
    plsc.subcore_barrier()

    for piece in range(2):
        pltpu.sync_copy(rcol_hbm.at[s, piece], col_t.at[pl.ds(0, 42)])
        pltpu.sync_copy(rew_hbm.at[s, piece], ew_t.at[pl.ds(0, 42)])

        def deg_j(j, carry):
            pltpu.sync_copy(ew_t.at[j], deg_sh.at[col_t.at[j]], add=True)
            return carry
        lax.fori_loop(0, 42, deg_j, 0)
    plsc.subcore_barrier()

    pltpu.sync_copy(deg_sh.at[pl.ds(s * _RT, _RT)], dbuf)

    def newt(i, carry):
        v = dbuf[pl.ds(i * 16, 16)]
        yi = jnp.int32(0x5F3759DF) - (
            lax.bitcast_convert_type(v, jnp.int32) >> 1)
        y = lax.bitcast_convert_type(yi, f32)
        for _ in range(3):
            y = y * (1.5 - 0.5 * v * y * y)
        dvbuf[pl.ds(i * 16, 16)] = y
        return carry
    lax.fori_loop(0, _RT // 16, newt, 0)
    pltpu.sync_copy(dvbuf, dinv_sh.at[pl.ds(s * _RT, _RT)])
    plsc.subcore_barrier()
    pltpu.sync_copy(dinv_sh, dinv_t)

    pltpu.sync_copy(brow_hbm.at[wid], row_t)
    pltpu.sync_copy(bcol_hbm.at[wid], col_t)
    pltpu.sync_copy(bew_hbm.at[wid], ew_t)

    lo_w = wid * _DR

    def norm_j(j, carry):
        for k in range(8):
            rv = row_t[j, pl.ds(k * 16, 16)]
            cv = col_t[j, pl.ds(k * 16, 16)]
            ev = ew_t[j, pl.ds(k * 16, 16)]
            nv = (plsc.load_gather(dinv_t, [rv]) * ev
                  * plsc.load_gather(dinv_t, [cv]))
            norm_t[j, pl.ds(k * 16, 16)] = nv
            col_t[j, pl.ds(k * 16, 16)] = (cv - lo_w) * _HG
        return carry
    lax.fori_loop(0, _NB_B, norm_j, 0)

    def t_body(t, carry):
        def zcp(m, carry2):
            acc_t[pl.ds(m * 16, 16)] = zeros16
            return carry2
        lax.fori_loop(0, _DR * _HG // 16, zcp, 0)

        base = t * _NP

        def gj(j, carry2):
            for k in range(8):
                gidx_t[j, pl.ds(k * 16, 16)] = (
                    row_t[j, pl.ds(k * 16, 16)] + base)
            return carry2
        lax.fori_loop(0, _NB_B, gj, 0)

        pltpu.async_copy(xw_hbm.at[gidx_t.at[0]], buf0, sem0)

        def pair(jj, carry2):
            j = jj * 2
            for b in range(2):
                jb = j + b
                nb = jb + 1

                @pl.when(nb < _NB_B)
                def _():
                    pltpu.async_copy(xw_hbm.at[gidx_t.at[nb]],
                                     bufs[(b + 1) % 2], sems[(b + 1) % 2])

                pltpu.make_async_copy(xw_hbm.at[gidx_t.at[jb]],
                                      bufs[b], sems[b]).wait()
                buf = bufs[b]

                def accum_q(qq, carry3):
                    nv = norm_t[jb, pl.ds(qq * 16, 16)]
                    lv = col_t[jb, pl.ds(qq * 16, 16)]
                    for l in range(16):
                        e = qq * 16 + l
                        il = jnp.full((16,), l, jnp.int32)
                        svv = nv.at[il].get(mode='promise_in_bounds')
                        bases = lv.at[il].get(mode='promise_in_bounds')
                        addr0 = bases + iota16
                        for k in range(8):
                            plsc.addupdate_scatter(
                                acc_t, [addr0 + (k * 16)],
                                buf[e, pl.ds(k * 16, 16)] * svv)
                    return carry3
                lax.fori_loop(0, _G // 16, accum_q, 0)
            return carry2
        lax.fori_loop(0, _NB_B // 2, pair, 0)

        pltpu.sync_copy(acc_t, acc_hbm.at[t, wid])
        return carry
    lax.fori_loop(0, _T, t_body, 0)


def _sc_prop(xwflat, rcol4, rew4, brow3, bcol3, bew3):
    mesh = plsc.VectorSubcoreMesh(core_axis_name="c", subcore_axis_name="s",
                                  num_cores=2, num_subcores=_TILES)
    f = pl.kernel(
        _sc_body, mesh=mesh,
        compiler_params=pltpu.CompilerParams(needs_layout_passes=False),
        out_type=jax.ShapeDtypeStruct((_T, _W, _DR * _HG), jnp.float32),
        scratch_types=[
            pltpu.VMEM((_NB_B, _G), jnp.int32),
            pltpu.VMEM((_NB_B, _G), jnp.int32),
            pltpu.VMEM((_NB_B, _G), jnp.float32),
            pltpu.VMEM((_NB_B, _G), jnp.float32),
            pltpu.VMEM((_NB_B, _G), jnp.int32),
            pltpu.VMEM((_RT,), jnp.float32),
            pltpu.VMEM((_RT,), jnp.float32),
            pltpu.VMEM((_NP,), jnp.float32),
            pltpu.VMEM((_DR * _HG,), jnp.float32),
            pltpu.VMEM((_G, _HG), jnp.float32),
            pltpu.VMEM((_G, _HG), jnp.float32),
            pltpu.VMEM((32, _HG), jnp.float32),
            pltpu.VMEM_SHARED((_NP,), jnp.float32),
            pltpu.VMEM_SHARED((_NP,), jnp.float32),
            pltpu.SemaphoreType.DMA,
            pltpu.SemaphoreType.DMA,
        ],
    )
    return f(xwflat, rcol4, rew4, brow3, bcol3, bew3)


def _tail_body(acc_ref, batch_ref, bg_ref, wcls_ref, bcls_ref, out_ref,
               pooled, counts):
    i = pl.program_id(0)

    @pl.when(i == 0)
    def _():
        pooled[...] = jnp.zeros_like(pooled)
        counts[...] = jnp.zeros_like(counts)
        out_ref[...] = jnp.zeros_like(out_ref)

    acc = acc_ref[...].astype(jnp.float32)
    bg = bg_ref[...][0:1, :]
    feats = jax.nn.relu(acc + bg[None])
    node = jnp.mean(feats, axis=0)
    bvec = batch_ref[...].reshape(1, _RB)
    ohT = (jax.lax.broadcasted_iota(jnp.int32, (_B, _RB), 0)
           == bvec).astype(jnp.float32)
    pooled[...] += jnp.dot(ohT, node, preferred_element_type=jnp.float32)
    counts[...] += jnp.broadcast_to(
        jnp.sum(ohT, axis=1, keepdims=True), (_B, _HG))

    @pl.when(i == _NBLK - 1)
    def _():
        g = pooled[...] / jnp.maximum(counts[...], 1.0)
        out_ref[...] = (jnp.dot(g, wcls_ref[...],
                                preferred_element_type=jnp.float32)
                        + bcls_ref[...][0:1, :])


def _tail(acc, batch3, bg8, W_cls, bcls8):
    full = lambda s: pl.BlockSpec(s, lambda i: (0,) * len(s))
    return pl.pallas_call(
        _tail_body,
        grid=(_NBLK,),
        in_specs=[
            pl.BlockSpec((_T, _RB, _HG), lambda i: (0, i, 0)),
            pl.BlockSpec((1, 1, _RB), lambda i: (i, 0, 0)),
            full((8, _HG)), full((_HG, 3)), full((8, 3)),
        ],
        out_specs=full((_B, 3)),
        out_shape=jax.ShapeDtypeStruct((_B, 3), jnp.float32),
        scratch_shapes=[
            pltpu.VMEM((_B, _HG), jnp.float32),
            pltpu.VMEM((_B, _HG), jnp.float32),
        ],
        compiler_params=pltpu.CompilerParams(
            dimension_semantics=("arbitrary",)),
    )(acc, batch3, bg8, W_cls, bcls8)


def kernel(x, edge_index, edge_weight, batch,
           Wih_f, Whh_f, bih_f, bhh_f,
           Wih_r, Whh_r, bih_r, bhh_r,
           W_gcn, b_gcn, W_cls, b_cls):
    xw = _lstm_xw(x, Wih_f, Whh_f, bih_f, bhh_f,
                  Wih_r, Whh_r, bih_r, bhh_r, W_gcn)
    xwflat = xw.reshape(_T * _NP, _HG)

    row = edge_index[0]
    col = edge_index[1]
    loop_idx = jnp.arange(_N, dtype=jnp.int32)
    pad = _EP - _E - _N
    row0 = jnp.concatenate([row, loop_idx])
    col0 = jnp.concatenate([col, loop_idx])
    ew0 = jnp.concatenate([edge_weight, jnp.ones((_N,), jnp.float32)])
    cola = jnp.concatenate([col0, jnp.zeros((pad,), jnp.int32)]).reshape(
        _TILES, 2, _EPT // (2 * _G), _G)
    ewa = jnp.concatenate([ew0, jnp.zeros((pad,), jnp.float32)]).reshape(
        _TILES, 2, _EPT // (2 * _G), _G)
    nb = col0 // _DR
    order = jnp.argsort(nb, stable=True)
    nb_s = nb[order]
    offs = jnp.cumsum(jnp.bincount(nb, length=_W)) - jnp.bincount(
        nb, length=_W)
    pos = jnp.arange(row0.shape[0], dtype=jnp.int32) - offs[nb_s]
    dest = nb_s * _CAP + pos
    slots = jnp.arange(_W * _CAP, dtype=jnp.int32)
    browb = jnp.zeros((_W * _CAP,), jnp.int32).at[dest].set(
        row0[order], mode='drop').reshape(_W, _NB_B, _G)
    bcolb = ((slots // _CAP) * _DR).at[dest].set(
        col0[order], mode='drop').reshape(_W, _NB_B, _G)
    bewb = jnp.zeros((_W * _CAP,), jnp.float32).at[dest].set(
        ew0[order], mode='drop').reshape(_W, _NB_B, _G)

    acc = _sc_prop(xwflat, cola, ewa, browb, bcolb, bewb).reshape(
        _T, _NP, _HG)

    batch3 = jnp.pad(batch, (0, _NP - _N),
                     constant_values=_B).reshape(_NBLK, 1, _RB)
    bg8 = jnp.broadcast_to(b_gcn[None, :], (8, _HG))
    bcls8 = jnp.broadcast_to(b_cls[None, :], (8, 3))
    return _tail(acc, batch3, bg8, W_cls, bcls8)
